# baseline (speedup 1.0000x reference)
import functools

import jax
import jax.numpy as jnp
from jax.experimental import pallas as pl

_NHID = 128
_ALPHA = 0.2


def _mm_kernel(x_ref, w_ref, o_ref):
    o_ref[...] = jnp.dot(x_ref[...], w_ref[...],
                         preferred_element_type=jnp.float32)


def _mm(x, w, bm=512):
    m, k = x.shape
    n = w.shape[1]
    grid = (pl.cdiv(m, bm),)
    return pl.pallas_call(
        _mm_kernel,
        grid=grid,
        in_specs=[pl.BlockSpec((bm, k), lambda i: (i, 0)),
                  pl.BlockSpec((k, n), lambda i: (0, 0))],
        out_specs=pl.BlockSpec((bm, n), lambda i: (i, 0)),
        out_shape=jax.ShapeDtypeStruct((m, n), jnp.float32),
    )(x, w)


def _leaky(x):
    return jnp.where(x > 0, x, _ALPHA * x)


def _elu(x):
    return jnp.where(x > 0, x, jnp.expm1(x))


def _edge_pass(P1, P2, P3, s1, s2, s3, src, dst):
    pw = -_leaky(s1[src] + s2[dst] + s3)
    ee = jnp.exp(pw)
    em = P1[src] + P2[dst] + P3
    ew = ee[:, None] * em
    h1 = jax.ops.segment_sum(ew, src, num_segments=P1.shape[0])
    r1 = jax.ops.segment_sum(ee, src, num_segments=P1.shape[0])
    h2 = jax.ops.segment_sum(ew, dst, num_segments=P2.shape[0])
    r2 = jax.ops.segment_sum(ee, dst, num_segments=P2.shape[0])
    return h1, r1, h2, r2


def _norm(h, r):
    r = jnp.where(r == 0, 1e-12, r)
    return h / r[:, None]


def kernel(Corpus_, batch_inputs, entity_embeddings, relation_embed,
           type_embed, edge_list, edge_type, edge_embed, edge_list_nhop,
           a0, a2_0, a1, a2_1, W, a_out, a2_out):
    del Corpus_, batch_inputs, edge_list_nhop
    f1 = entity_embeddings.shape[1]
    f2 = type_embed.shape[1]
    dst = edge_list[0]
    src = edge_list[1]

    wa = []
    for a, a2 in ((a0, a2_0), (a1, a2_1)):
        a_s, a_t, a_e = a[:, :f1], a[:, f1:f1 + f2], a[:, f1 + f2:]
        wa.append((a_s, a_t, a_e, a2))

    cat_s = jnp.concatenate([wa[0][0].T, wa[1][0].T], axis=1)
    cat_t = jnp.concatenate([wa[0][1].T, wa[1][1].T], axis=1)
    cat_e = jnp.concatenate([wa[0][2].T, wa[1][2].T], axis=1)
    P1 = _mm(entity_embeddings, cat_s)
    P2 = _mm(type_embed, cat_t)
    P3 = _mm(edge_embed, cat_e)

    xs1 = []
    xs2 = []
    for h in range(2):
        a2 = wa[h][3]
        sl = slice(h * _NHID, (h + 1) * _NHID)
        P1h, P2h, P3h = P1[:, sl], P2[:, sl], P3[:, sl]
        s1 = P1h @ a2[0]
        s2 = P2h @ a2[0]
        s3 = P3h @ a2[0]
        h1, r1, h2, r2 = _edge_pass(P1h, P2h, P3h, s1, s2, s3, src, dst)
        xs1.append(_elu(_norm(h1, r1)))
        xs2.append(_elu(_norm(h2, r2)))
    x1 = jnp.concatenate(xs1, axis=1)
    x2 = jnp.concatenate(xs2, axis=1)

    out_relation_1 = _mm(relation_embed, W)
    d = x1.shape[1]
    A_s, A_t, A_e = a_out[:, :d], a_out[:, d:2 * d], a_out[:, 2 * d:]
    Q1 = _mm(x1, A_s.T)
    Q2 = _mm(x2, A_t.T)
    Qr = _mm(out_relation_1, A_e.T)
    t1 = Q1 @ a2_out[0]
    t2 = Q2 @ a2_out[0]
    tr = Qr @ a2_out[0]
    h1, r1, h2, r2 = _edge_pass(Q1, Q2, Qr[edge_type],
                                t1, t2, tr[edge_type], src, dst)
    y1 = _elu(_norm(h1, r1))
    y2 = _elu(_norm(h2, r2))
    return y1, y2, out_relation_1


# speedup vs baseline: 6.1014x; 6.1014x over previous
import functools

import jax
import jax.numpy as jnp
from jax import lax
from jax.experimental import pallas as pl
from jax.experimental.pallas import tpu as pltpu
from jax.experimental.pallas import tpu_sc as plsc

_NHID = 128
_ALPHA = 0.2
_ROWW = 144
_CH = 128
_NSC = 2
_NSUB = 16
_NW = _NSC * _NSUB


def _mm_kernel(x_ref, w_ref, o_ref):
    o_ref[...] = jnp.dot(x_ref[...], w_ref[...],
                         preferred_element_type=jnp.float32)


def _mm(x, w, bm=1024):
    m, k = x.shape
    n = w.shape[1]
    return pl.pallas_call(
        _mm_kernel,
        grid=(pl.cdiv(m, bm),),
        in_specs=[pl.BlockSpec((bm, k), lambda i: (i, 0)),
                  pl.BlockSpec((k, n), lambda i: (0, 0))],
        out_specs=pl.BlockSpec((bm, n), lambda i: (i, 0)),
        out_shape=jax.ShapeDtypeStruct((m, n), jnp.float32),
    )(x, w)


def _combine_kernel(pa_ref, pb_ref, o_ref):
    for idx, p in ((slice(0, _NHID), pa_ref), (slice(_NHID, 2 * _NHID),
                                               pb_ref)):
        h = p[0, :, :_NHID] + p[1, :, :_NHID]
        r = p[0, :, _NHID:_NHID + 1] + p[1, :, _NHID:_NHID + 1]
        r = jnp.where(r == 0, 1e-12, r)
        x = h / r
        o_ref[:, idx] = jnp.where(x > 0, x, jnp.exp(x) - 1.0)


def _combine(pa, pb, bm=512):
    n = pa.shape[1]
    return pl.pallas_call(
        _combine_kernel,
        grid=(pl.cdiv(n, bm),),
        in_specs=[pl.BlockSpec((2, bm, _ROWW), lambda i: (0, i, 0)),
                  pl.BlockSpec((2, bm, _ROWW), lambda i: (0, i, 0))],
        out_specs=pl.BlockSpec((bm, 2 * _NHID), lambda i: (i, 0)),
        out_shape=jax.ShapeDtypeStruct((n, 2 * _NHID), jnp.float32),
    )(pa, pb)


@functools.cache
def _make_edge_pass(n1, n2p, n3, ne):
    nchunks = ne // _CH
    rows1_per_tile = n1 // _NSUB
    z1 = 128
    nz1 = rows1_per_tile // z1
    rows2_per_tile = n2p // _NSUB
    mesh = plsc.VectorSubcoreMesh(core_axis_name="c", subcore_axis_name="s")

    @functools.partial(
        pl.kernel,
        out_type=(jax.ShapeDtypeStruct((_NSC, n1, _ROWW), jnp.float32),
                  jax.ShapeDtypeStruct((_NSC, n2p, _ROWW), jnp.float32)),
        mesh=mesh,
        scratch_types=[
            pltpu.VMEM((128,), jnp.int32),
            pltpu.VMEM((128,), jnp.int32),
            pltpu.VMEM((128,), jnp.int32),
            pltpu.VMEM((_CH, _ROWW), jnp.float32),
            pltpu.VMEM_SHARED((n1, _ROWW), jnp.float32),
            pltpu.VMEM_SHARED((n2p, _ROWW), jnp.float32),
        ],
        compiler_params=pltpu.CompilerParams(use_tc_tiling_on_sc=False,
                                             needs_layout_passes=False),
    )
    def edge_pass(src_h, dst_h, i3_h, t1_h, t2_h, t3_h, out1_h, out2_h,
                  src_v, dst_v, i3_v, buf, h1_sh, h2_sh):
        c = lax.axis_index("c")
        s = lax.axis_index("s")
        g = c * _NSUB + s

        def zrow(r, _):
            for j in range(_ROWW // 16):
                buf[r, pl.ds(16 * j, 16)] = jnp.zeros((16,), jnp.float32)
            return _
        lax.fori_loop(0, _CH, zrow, None)
        for k in range(nz1):
            pltpu.sync_copy(
                buf.at[pl.ds(0, z1)],
                h1_sh.at[pl.ds(s * rows1_per_tile + k * z1, z1)])
        pltpu.sync_copy(buf.at[pl.ds(0, rows2_per_tile)],
                        h2_sh.at[pl.ds(s * rows2_per_tile, rows2_per_tile)])
        plsc.subcore_barrier()

        lo = nchunks * g // _NW
        hi = nchunks * (g + 1) // _NW

        def chunk_body(ci, _):
            pltpu.sync_copy(src_h.at[ci], src_v)
            pltpu.sync_copy(dst_h.at[ci], dst_v)
            pltpu.sync_copy(i3_h.at[ci], i3_v)
            pltpu.sync_copy(t1_h.at[src_v], buf)
            pltpu.sync_copy(t2_h.at[dst_v], buf, add=True)
            pltpu.sync_copy(t3_h.at[i3_v], buf, add=True)

            def row_body(e, _):
                pw = jnp.full((16,), buf[e, pl.ds(_NHID, 16)][0],
                              jnp.float32)
                eev = jnp.exp(-jnp.where(pw > 0, pw, _ALPHA * pw))
                for j in range(_NHID // 16):
                    sl = pl.ds(16 * j, 16)
                    buf[e, sl] = buf[e, sl] * eev
                buf[e, pl.ds(_NHID, 16)] = eev
                return _
            lax.fori_loop(0, _CH, row_body, None)

            pltpu.sync_copy(buf, h1_sh.at[src_v], add=True)
            pltpu.sync_copy(buf, h2_sh.at[dst_v], add=True)
            return _

        lax.fori_loop(lo, hi, chunk_body, None)
        plsc.subcore_barrier()

        for k in range(nz1):
            off = s * rows1_per_tile + k * z1
            pltpu.sync_copy(h1_sh.at[pl.ds(off, z1)],
                            out1_h.at[c, pl.ds(off, z1)])
        pltpu.sync_copy(h2_sh.at[pl.ds(s * rows2_per_tile, rows2_per_tile)],
                        out2_h.at[c, pl.ds(s * rows2_per_tile,
                                           rows2_per_tile)])

    return edge_pass


def _ext_w(wt, u):
    k = wt.shape[0]
    return jnp.concatenate(
        [wt, u[:, None], jnp.zeros((k, _ROWW - _NHID - 1), jnp.float32)],
        axis=1)


def kernel(Corpus_, batch_inputs, entity_embeddings, relation_embed,
           type_embed, edge_list, edge_type, edge_embed, edge_list_nhop,
           a0, a2_0, a1, a2_1, W, a_out, a2_out):
    del Corpus_, batch_inputs, edge_list_nhop
    f1 = entity_embeddings.shape[1]
    f2 = type_embed.shape[1]
    ne = edge_type.shape[0]
    n_ent = entity_embeddings.shape[0]
    n_typ = type_embed.shape[0]
    n1p = 10240
    n2p = 512

    nchunks = ne // _CH
    dst3 = edge_list[0].reshape(nchunks, _CH)
    src3 = edge_list[1].reshape(nchunks, _CH)
    et3 = edge_type.reshape(nchunks, _CH)
    io3 = jnp.arange(ne, dtype=jnp.int32).reshape(nchunks, _CH)

    l1_pass = _make_edge_pass(n1p, n2p, ne, ne)
    parts = []
    for a, a2 in ((a0, a2_0), (a1, a2_1)):
        a_s, a_t, a_e = a[:, :f1], a[:, f1:f1 + f2], a[:, f1 + f2:]
        t1 = _mm(entity_embeddings, _ext_w(a_s.T, a_s.T @ a2[0]))
        t2 = _mm(type_embed, _ext_w(a_t.T, a_t.T @ a2[0]))
        t3 = _mm(edge_embed, _ext_w(a_e.T, a_e.T @ a2[0]))
        parts.append(l1_pass(src3, dst3, io3, t1, t2, t3))
    x1 = _combine(parts[0][0], parts[1][0])[:n_ent]
    x2 = _combine(parts[0][1], parts[1][1])[:n_typ]

    out_relation_1 = _mm(relation_embed, W)
    d = 2 * _NHID
    A_s, A_t, A_e = a_out[:, :d], a_out[:, d:2 * d], a_out[:, 2 * d:]
    u_s = A_s.T @ a2_out[0]
    u_t = A_t.T @ a2_out[0]
    u_e = A_e.T @ a2_out[0]
    out_pass = _make_edge_pass(n1p, n2p, out_relation_1.shape[0], ne)
    parts = []
    for h in range(2):
        cols = slice(h * _NHID, (h + 1) * _NHID)
        t1 = _mm(x1, _ext_w(A_s.T[:, cols], u_s))
        t2 = _mm(x2, _ext_w(A_t.T[:, cols], u_t))
        t3 = _mm(out_relation_1, _ext_w(A_e.T[:, cols], u_e))
        parts.append(out_pass(src3, dst3, et3, t1, t2, t3))
    y1 = _combine(parts[0][0], parts[1][0])[:n_ent]
    y2 = _combine(parts[0][1], parts[1][1])[:n_typ]
    return y1, y2, out_relation_1
